# baseline (speedup 1.0000x reference)
import functools

import jax
import jax.numpy as jnp
from jax import lax
from jax.experimental import pallas as pl
from jax.experimental.pallas import tpu as pltpu
from jax.experimental.pallas import tpu_sc as plsc

_K = 8
_NEG = float("-inf")


def _scores_body(x_ref, w_ref, b_ref, sc_ref):
    j = pl.program_id(1)
    p_tile = w_ref.shape[0]
    s = jax.lax.dot_general(
        x_ref[...], w_ref[...], (((1,), (1,)), ((), ())),
        preferred_element_type=jnp.float32,
        precision=jax.lax.Precision.DEFAULT,
    )
    sc_ref[...] = s + b_ref[pl.ds(j * p_tile, p_tile)][None, :]


def _tc_scores(x2d, w, b):
    t, d = x2d.shape
    p = w.shape[0]
    t_tile = min(512, t)
    p_tile = min(512, p)
    return pl.pallas_call(
        _scores_body,
        grid=(t // t_tile, p // p_tile),
        in_specs=[
            pl.BlockSpec((t_tile, d), lambda i, j: (i, 0)),
            pl.BlockSpec((p_tile, d), lambda i, j: (j, 0)),
            pl.BlockSpec((p,), lambda i, j: (0,)),
        ],
        out_specs=pl.BlockSpec((t_tile, p_tile), lambda i, j: (i, j)),
        out_shape=jax.ShapeDtypeStruct((t, p), jnp.float32),
        compiler_params=pltpu.CompilerParams(
            dimension_semantics=("parallel", "arbitrary"),
        ),
    )(x2d, w, b)


_SUPER = 16


def _sc_topk(scores):
    t, p = scores.shape
    info = plsc.get_sparse_core_info()
    nc = info.num_cores
    nw = nc * info.num_subcores
    rows_w = t // nw
    n_super = p // (16 * _SUPER)

    mesh = plsc.VectorSubcoreMesh(core_axis_name="c", subcore_axis_name="s")

    @functools.partial(
        pl.kernel, mesh=mesh,
        compiler_params=pltpu.CompilerParams(needs_layout_passes=False),
        out_type=jax.ShapeDtypeStruct((t * _K,), jnp.int32),
        scratch_types=[
            pltpu.VMEM((p,), jnp.float32),
            pltpu.VMEM((p,), jnp.float32),
            pltpu.VMEM((rows_w * _K + 16,), jnp.int32),
            pltpu.SemaphoreType.DMA,
            pltpu.SemaphoreType.DMA,
        ],
    )
    def k(scores_hbm, idx_hbm, buf0, buf1, outbuf, sem0, sem1):
        wid = lax.axis_index("s") * nc + lax.axis_index("c")
        base = wid * rows_w
        lane = lax.broadcasted_iota(jnp.int32, (16,), 0)

        def merge_sorted(sv, si, carry):
            cv, ci, _ = carry
            sel = cv >= sv
            nv = jnp.where(sel, cv, sv)
            ni = jnp.where(sel, ci, si)
            cv, ci = plsc.sort_key_val(nv, ni, descending=False)
            return cv, ci, cv[0]

        def scan_row(buf, row_local):
            def sc_body(q, carry):
                om = buf[pl.ds(q * (16 * _SUPER), 16)]
                for u in range(1, _SUPER):
                    om = jnp.maximum(om, buf[pl.ds(q * (16 * _SUPER) + u * 16, 16)])

                def do_merge(c0):
                    c = c0
                    for u in range(_SUPER):
                        col0 = q * (16 * _SUPER) + u * 16
                        v = buf[pl.ds(col0, 16)]
                        hit = plsc.all_reduce_population_count(v > c[2])[0]

                        def m(a, vv=v, cc=col0):
                            sv, si = plsc.sort_key_val(vv, cc + lane,
                                                       descending=True)
                            return merge_sorted(sv, si, a)

                        c = lax.cond(hit > 0, m, lambda a: a, c)
                    return c

                any_hit = plsc.all_reduce_population_count(om > carry[2])[0]
                return lax.cond(any_hit > 0, do_merge, lambda a: a, carry)

            init = (jnp.full((16,), _NEG, jnp.float32),
                    jnp.zeros((16,), jnp.int32),
                    jnp.float32(_NEG))
            cv, ci, _ = lax.fori_loop(0, n_super, sc_body, init)
            sv, si = plsc.sort_key_val(cv, ci, descending=True)
            outbuf[pl.ds(row_local * _K, 16)] = si

        pltpu.async_copy(scores_hbm.at[base], buf0, sem0)

        def pair_body(i, _):
            row_e = 2 * i
            row_o = row_e + 1
            pltpu.async_copy(scores_hbm.at[base + row_o], buf1, sem1)
            pltpu.make_async_copy(scores_hbm.at[base + row_e], buf0,
                                  sem0).wait()
            scan_row(buf0, row_e)

            @pl.when(row_e + 2 < rows_w)
            def _():
                pltpu.async_copy(scores_hbm.at[base + row_e + 2], buf0, sem0)

            pltpu.make_async_copy(scores_hbm.at[base + row_o], buf1,
                                  sem1).wait()
            scan_row(buf1, row_o)

            @pl.when(row_o + 2 < rows_w)
            def _():
                pltpu.async_copy(scores_hbm.at[base + row_o + 2], buf1, sem1)

            return 0

        lax.fori_loop(0, rows_w // 2, pair_body, 0)
        pltpu.sync_copy(outbuf.at[pl.ds(0, rows_w * _K)],
                        idx_hbm.at[pl.ds(base * _K, rows_w * _K)])

    return k(scores).reshape(t, _K)


@jax.jit
def _router(x2d, w, b):
    scores = _tc_scores(x2d, w, b)
    idx = _sc_topk(scores)
    return idx, scores


def kernel(x, w, b):
    bsz, seq, d = x.shape
    p = w.shape[0]
    x2d = x.reshape(bsz * seq, d)
    idx_out, scores = _router(x2d, w, b)
    return idx_out.reshape(bsz, seq, _K), scores.reshape(bsz, seq, p)
